# Initial kernel scaffold; baseline (speedup 1.0000x reference)
import jax
import jax.numpy as jnp
from jax.experimental import pallas as pl


def kernel(features, edge_index, edge_weight, W, b):
    raise NotImplementedError("write your pallas kernel here")


# speedup vs baseline: 4.2766x; 4.2766x over previous
import dataclasses
import functools

import jax
import jax.numpy as jnp
from jax import lax
from jax.experimental import pallas as pl
from jax.experimental.pallas import tpu as pltpu
from jax.experimental.pallas import tpu_sc as plsc

NC = 2
NS = 16
DH = 64
VPR = DH // 16


def _sc_aggregate(feat2, src3d, dst3d, w3d, n):
    nch, chunk = src3d.shape[1:]
    zrows = chunk * 8
    npad = NS * zrows
    assert npad >= n

    mesh = plsc.VectorSubcoreMesh(
        core_axis_name="c", subcore_axis_name="s", num_cores=NC, num_subcores=NS
    )
    cp = pltpu.CompilerParams()
    if "needs_layout_passes" in pltpu.CompilerParams.__dataclass_fields__:
        cp = dataclasses.replace(cp, needs_layout_passes=False)
    if "use_tc_tiling_on_sc" in pltpu.CompilerParams.__dataclass_fields__:
        cp = dataclasses.replace(cp, use_tc_tiling_on_sc=False)

    @functools.partial(
        pl.kernel,
        out_type=jax.ShapeDtypeStruct((NC, npad, DH), jnp.float32),
        mesh=mesh,
        compiler_params=cp,
        scratch_types=[
            pltpu.VMEM((nch, chunk), jnp.int32),
            pltpu.VMEM((nch, chunk), jnp.int32),
            pltpu.VMEM((nch, chunk), jnp.float32),
            pltpu.VMEM((chunk, DH), jnp.float32),
            pltpu.VMEM_SHARED((npad, DH), jnp.float32),
        ],
    )
    def agg_kernel(feat_hbm, src_hbm, dst_hbm, w_hbm, out_hbm,
                   src_v, dst_v, w_v, rows_v, acc):
        cid = lax.axis_index("c")
        sid = lax.axis_index("s")

        pltpu.sync_copy(src_hbm.at[sid], src_v)
        pltpu.sync_copy(dst_hbm.at[sid], dst_v)
        pltpu.sync_copy(w_hbm.at[sid], w_v)

        @pl.loop(0, nch)
        def _(r):
            for k in range(chunk // 16):
                sl = pl.ds(k * 16, 16)
                src_v[r, sl] = src_v[r, sl] * 2 + cid

        zero = jnp.zeros((16,), jnp.float32)

        @pl.loop(0, chunk)
        def _(r):
            for k in range(VPR):
                rows_v[r, pl.ds(k * 16, 16)] = zero

        for i in range(8):
            pltpu.sync_copy(rows_v, acc.at[pl.ds(sid * zrows + i * chunk, chunk)])
        plsc.subcore_barrier()

        @pl.loop(0, nch)
        def _(t):
            pltpu.sync_copy(feat_hbm.at[src_v.at[t]], rows_v)

            @pl.loop(0, chunk)
            def _(r):
                i0 = jnp.full((16,), t, jnp.int32)
                i1 = jnp.full((16,), r, jnp.int32)
                wj = plsc.load_gather(w_v, [i0, i1])
                for k in range(VPR):
                    sl = pl.ds(k * 16, 16)
                    rows_v[r, sl] = rows_v[r, sl] * wj

            pltpu.sync_copy(rows_v, acc.at[dst_v.at[t]], add=True)

        plsc.subcore_barrier()

        pltpu.sync_copy(
            acc.at[pl.ds(sid * zrows, zrows)],
            out_hbm.at[cid, pl.ds(sid * zrows, zrows)],
        )

    return agg_kernel(feat2, src3d, dst3d, w3d)


def _linear_body(p_ref, w_ref, b_ref, o_ref):
    y0 = lax.dot_general(
        p_ref[0], w_ref[:, 0:DH], (((1,), (1,)), ((), ())),
        preferred_element_type=jnp.float32,
    )
    y1 = lax.dot_general(
        p_ref[1], w_ref[:, DH:2 * DH], (((1,), (1,)), ((), ())),
        preferred_element_type=jnp.float32,
    )
    o_ref[...] = y0 + y1 + b_ref[...]


def _tc_linear(partials, W, b, n):
    nc, npad, dh = partials.shape
    d_out = W.shape[0]
    blk = 1000
    assert n % blk == 0
    return pl.pallas_call(
        _linear_body,
        grid=(n // blk,),
        in_specs=[
            pl.BlockSpec((nc, blk, dh), lambda i: (0, i, 0)),
            pl.BlockSpec((d_out, 2 * dh), lambda i: (0, 0)),
            pl.BlockSpec((1, d_out), lambda i: (0, 0)),
        ],
        out_specs=pl.BlockSpec((blk, d_out), lambda i: (i, 0)),
        out_shape=jax.ShapeDtypeStruct((n, d_out), jnp.float32),
    )(partials, W, b.reshape(1, d_out))


def kernel(features, edge_index, edge_weight, W, b):
    n, d = features.shape
    e = edge_weight.shape[0]
    assert d == 2 * DH
    dst = edge_index[0].astype(jnp.int32)
    src = edge_index[1].astype(jnp.int32)
    w = edge_weight.astype(jnp.float32)

    eps = e // NS
    assert eps * NS == e
    chunk = next(c for c in range(128, 7, -8) if eps % c == 0)
    nch = eps // chunk

    feat2 = features.reshape(2 * n, DH)
    src3d = src.reshape(NS, nch, chunk)
    dst3d = dst.reshape(NS, nch, chunk)
    w3d = w.reshape(NS, nch, chunk)

    partials = _sc_aggregate(feat2, src3d, dst3d, w3d, n)
    return _tc_linear(partials, W, b, n)


# speedup vs baseline: 7.3616x; 1.7214x over previous
import dataclasses
import functools

import jax
import jax.numpy as jnp
from jax import lax
from jax.experimental import pallas as pl
from jax.experimental.pallas import tpu as pltpu
from jax.experimental.pallas import tpu_sc as plsc

NC = 2
NS = 16
DH = 64
VPR = DH // 16


def _sc_aggregate(feat2, src3d, dst3d, w3d, n):
    nch, chunk = src3d.shape[1:]
    zrows = chunk * 8
    npad = NS * zrows
    assert npad >= n

    mesh = plsc.VectorSubcoreMesh(
        core_axis_name="c", subcore_axis_name="s", num_cores=NC, num_subcores=NS
    )
    cp = pltpu.CompilerParams()
    if "needs_layout_passes" in pltpu.CompilerParams.__dataclass_fields__:
        cp = dataclasses.replace(cp, needs_layout_passes=False)
    if "use_tc_tiling_on_sc" in pltpu.CompilerParams.__dataclass_fields__:
        cp = dataclasses.replace(cp, use_tc_tiling_on_sc=False)

    nbuf = 4
    main_end = max(0, ((nch - 2) // nbuf) * nbuf)

    @functools.partial(
        pl.kernel,
        out_type=jax.ShapeDtypeStruct((NC, npad, DH), jnp.float32),
        mesh=mesh,
        compiler_params=cp,
        scratch_types=[
            pltpu.VMEM((nch, chunk), jnp.int32),
            pltpu.VMEM((nch, chunk), jnp.int32),
            pltpu.VMEM((nch, chunk), jnp.float32),
            [pltpu.VMEM((chunk, DH), jnp.float32)] * nbuf,
            [pltpu.SemaphoreType.DMA] * nbuf,
            [pltpu.SemaphoreType.DMA] * nbuf,
            pltpu.VMEM_SHARED((npad, DH), jnp.float32),
        ],
    )
    def agg_kernel(feat_hbm, src_hbm, dst_hbm, w_hbm, out_hbm,
                   src_v, dst_v, w_v, bufs, gsems, ssems, acc):
        cid = lax.axis_index("c")
        sid = lax.axis_index("s")

        pltpu.sync_copy(src_hbm.at[sid], src_v)
        pltpu.sync_copy(dst_hbm.at[sid], dst_v)
        pltpu.sync_copy(w_hbm.at[sid], w_v)

        @pl.loop(0, nch)
        def _(r):
            for k in range(chunk // 16):
                sl = pl.ds(k * 16, 16)
                src_v[r, sl] = src_v[r, sl] * 2 + cid

        zero = jnp.zeros((16,), jnp.float32)

        @pl.loop(0, chunk)
        def _(r):
            for k in range(VPR):
                bufs[0][r, pl.ds(k * 16, 16)] = zero

        for i in range(8):
            pltpu.sync_copy(bufs[0], acc.at[pl.ds(sid * zrows + i * chunk, chunk)])
        plsc.subcore_barrier()

        def gather_start(t, j):
            pltpu.async_copy(feat_hbm.at[src_v.at[t]], bufs[j], gsems[j])

        def gather_wait(t, j):
            pltpu.make_async_copy(feat_hbm.at[src_v.at[t]], bufs[j], gsems[j]).wait()

        def scatter_start(t, j):
            pltpu.sync_copy(bufs[j], acc.at[dst_v.at[t]], add=True)

        def scatter_wait(t, j):
            pass

        def scale(t, j):
            buf = bufs[j]

            @pl.loop(0, chunk)
            def _(r):
                i0 = jnp.full((16,), t, jnp.int32)
                i1 = jnp.full((16,), r, jnp.int32)
                wj = plsc.load_gather(w_v, [i0, i1])
                for k in range(VPR):
                    sl = pl.ds(k * 16, 16)
                    buf[r, sl] = buf[r, sl] * wj

        gather_start(0, 0)
        gather_start(1, 1)

        @pl.loop(0, main_end, step=nbuf)
        def _(t0):
            for j in range(nbuf):
                t = t0 + j
                j2 = (j + 2) % nbuf

                @pl.when(t >= 2)
                def _():
                    scatter_wait(t - 2, j2)

                @pl.when(t + 2 < nch)
                def _():
                    gather_start(t + 2, j2)

                gather_wait(t, j)
                scale(t, j)
                scatter_start(t, j)

        for t in range(main_end, nch):
            j = t % nbuf
            if t >= 2:
                scatter_wait(t - 2, (t + 2) % nbuf)
            if t + 2 < nch:
                gather_start(t + 2, (t + 2) % nbuf)
            gather_wait(t, j)
            scale(t, j)
            scatter_start(t, j)

        scatter_wait(nch - 2, (nch - 2) % nbuf)
        scatter_wait(nch - 1, (nch - 1) % nbuf)
        plsc.subcore_barrier()

        pltpu.sync_copy(
            acc.at[pl.ds(sid * zrows, zrows)],
            out_hbm.at[cid, pl.ds(sid * zrows, zrows)],
        )

    return agg_kernel(feat2, src3d, dst3d, w3d)


def _linear_body(p_ref, w_ref, b_ref, o_ref):
    y0 = lax.dot_general(
        p_ref[0], w_ref[:, 0:DH], (((1,), (1,)), ((), ())),
        preferred_element_type=jnp.float32,
    )
    y1 = lax.dot_general(
        p_ref[1], w_ref[:, DH:2 * DH], (((1,), (1,)), ((), ())),
        preferred_element_type=jnp.float32,
    )
    o_ref[...] = y0 + y1 + b_ref[...]


def _tc_linear(partials, W, b, n):
    nc, npad, dh = partials.shape
    d_out = W.shape[0]
    blk = 1000
    assert n % blk == 0
    return pl.pallas_call(
        _linear_body,
        grid=(n // blk,),
        in_specs=[
            pl.BlockSpec((nc, blk, dh), lambda i: (0, i, 0)),
            pl.BlockSpec((d_out, 2 * dh), lambda i: (0, 0)),
            pl.BlockSpec((1, d_out), lambda i: (0, 0)),
        ],
        out_specs=pl.BlockSpec((blk, d_out), lambda i: (i, 0)),
        out_shape=jax.ShapeDtypeStruct((n, d_out), jnp.float32),
    )(partials, W, b.reshape(1, d_out))


def kernel(features, edge_index, edge_weight, W, b):
    n, d = features.shape
    e = edge_weight.shape[0]
    assert d == 2 * DH
    dst = edge_index[0].astype(jnp.int32)
    src = edge_index[1].astype(jnp.int32)
    w = edge_weight.astype(jnp.float32)

    eps = e // NS
    assert eps * NS == e
    chunk = next(c for c in range(128, 7, -8) if eps % c == 0)
    nch = eps // chunk

    feat2 = features.reshape(2 * n, DH)
    src3d = src.reshape(NS, nch, chunk)
    dst3d = dst.reshape(NS, nch, chunk)
    w3d = w.reshape(NS, nch, chunk)

    partials = _sc_aggregate(feat2, src3d, dst3d, w3d, n)
    return _tc_linear(partials, W, b, n)


# speedup vs baseline: 9.4180x; 1.2793x over previous
import dataclasses
import functools

import jax
import jax.numpy as jnp
from jax import lax
from jax.experimental import pallas as pl
from jax.experimental.pallas import tpu as pltpu
from jax.experimental.pallas import tpu_sc as plsc

NC = 2
NS = 16
DH = 64
VPR = DH // 16


def _sc_aggregate(feat2, src3d, dst3d, w3d, n):
    nch, chunk = src3d.shape[1:]
    zrows = chunk * 8
    npad = NS * zrows
    assert npad >= n

    mesh = plsc.VectorSubcoreMesh(
        core_axis_name="c", subcore_axis_name="s", num_cores=NC, num_subcores=NS
    )
    cp = pltpu.CompilerParams()
    if "needs_layout_passes" in pltpu.CompilerParams.__dataclass_fields__:
        cp = dataclasses.replace(cp, needs_layout_passes=False)
    if "use_tc_tiling_on_sc" in pltpu.CompilerParams.__dataclass_fields__:
        cp = dataclasses.replace(cp, use_tc_tiling_on_sc=False)

    nbuf = 4
    main_end = max(0, ((nch - 2) // nbuf) * nbuf)

    @functools.partial(
        pl.kernel,
        out_type=jax.ShapeDtypeStruct((NC, npad, DH), jnp.float32),
        mesh=mesh,
        compiler_params=cp,
        scratch_types=[
            pltpu.VMEM((nch, chunk), jnp.int32),
            pltpu.VMEM((nch, chunk), jnp.int32),
            pltpu.VMEM((nch, chunk), jnp.float32),
            [pltpu.VMEM((chunk, DH), jnp.float32)] * nbuf,
            [pltpu.SemaphoreType.DMA] * nbuf,
            [pltpu.SemaphoreType.DMA] * nbuf,
            pltpu.VMEM_SHARED((npad, DH), jnp.float32),
        ],
    )
    def agg_kernel(feat_hbm, src_hbm, dst_hbm, w_hbm, out_hbm,
                   src_v, dst_v, w_v, bufs, gsems, ssems, acc):
        cid = lax.axis_index("c")
        sid = lax.axis_index("s")

        pltpu.sync_copy(src_hbm.at[sid], src_v)
        pltpu.sync_copy(dst_hbm.at[sid], dst_v)
        pltpu.sync_copy(w_hbm.at[sid], w_v)

        @pl.loop(0, nch)
        def _(r):
            for k in range(chunk // 16):
                sl = pl.ds(k * 16, 16)
                src_v[r, sl] = src_v[r, sl] * 2 + cid

        zero = jnp.zeros((16,), jnp.float32)

        @pl.loop(0, chunk)
        def _(r):
            for k in range(VPR):
                bufs[0][r, pl.ds(k * 16, 16)] = zero

        for i in range(8):
            pltpu.sync_copy(bufs[0], acc.at[pl.ds(sid * zrows + i * chunk, chunk)])
        plsc.subcore_barrier()

        def gather_start(t, j):
            pltpu.async_copy(feat_hbm.at[src_v.at[t]], bufs[j], gsems[j])

        def gather_wait(t, j):
            pltpu.make_async_copy(feat_hbm.at[src_v.at[t]], bufs[j], gsems[j]).wait()

        def scatter_start(t, j):
            pltpu.async_copy(bufs[j], acc.at[dst_v.at[t]], ssems[j], add=True)

        def scatter_wait(t, j):
            pltpu.make_async_copy(bufs[j], acc.at[dst_v.at[t]], ssems[j]).wait()

        def scale(t, j):
            buf = bufs[j]

            @pl.loop(0, chunk)
            def _(r):
                i0 = jnp.full((16,), t, jnp.int32)
                i1 = jnp.full((16,), r, jnp.int32)
                wj = plsc.load_gather(w_v, [i0, i1])
                for k in range(VPR):
                    sl = pl.ds(k * 16, 16)
                    buf[r, sl] = buf[r, sl] * wj

        gather_start(0, 0)
        gather_start(1, 1)

        @pl.loop(0, main_end, step=nbuf)
        def _(t0):
            for j in range(nbuf):
                t = t0 + j
                j2 = (j + 2) % nbuf

                @pl.when(t >= 2)
                def _():
                    scatter_wait(t - 2, j2)

                @pl.when(t + 2 < nch)
                def _():
                    gather_start(t + 2, j2)

                gather_wait(t, j)
                scale(t, j)
                scatter_start(t, j)

        for t in range(main_end, nch):
            j = t % nbuf
            if t >= 2:
                scatter_wait(t - 2, (t + 2) % nbuf)
            if t + 2 < nch:
                gather_start(t + 2, (t + 2) % nbuf)
            gather_wait(t, j)
            scale(t, j)
            scatter_start(t, j)

        scatter_wait(nch - 2, (nch - 2) % nbuf)
        scatter_wait(nch - 1, (nch - 1) % nbuf)
        plsc.subcore_barrier()

        pltpu.sync_copy(
            acc.at[pl.ds(sid * zrows, zrows)],
            out_hbm.at[cid, pl.ds(sid * zrows, zrows)],
        )

    return agg_kernel(feat2, src3d, dst3d, w3d)


def _linear_body(p_ref, w_ref, b_ref, o_ref):
    y0 = lax.dot_general(
        p_ref[0], w_ref[:, 0:DH], (((1,), (1,)), ((), ())),
        preferred_element_type=jnp.float32,
    )
    y1 = lax.dot_general(
        p_ref[1], w_ref[:, DH:2 * DH], (((1,), (1,)), ((), ())),
        preferred_element_type=jnp.float32,
    )
    o_ref[...] = y0 + y1 + b_ref[...]


def _tc_linear(partials, W, b, n):
    nc, npad, dh = partials.shape
    d_out = W.shape[0]
    blk = 1000
    assert n % blk == 0
    return pl.pallas_call(
        _linear_body,
        grid=(n // blk,),
        in_specs=[
            pl.BlockSpec((nc, blk, dh), lambda i: (0, i, 0)),
            pl.BlockSpec((d_out, 2 * dh), lambda i: (0, 0)),
            pl.BlockSpec((1, d_out), lambda i: (0, 0)),
        ],
        out_specs=pl.BlockSpec((blk, d_out), lambda i: (i, 0)),
        out_shape=jax.ShapeDtypeStruct((n, d_out), jnp.float32),
    )(partials, W, b.reshape(1, d_out))


def kernel(features, edge_index, edge_weight, W, b):
    n, d = features.shape
    e = edge_weight.shape[0]
    assert d == 2 * DH
    dst = edge_index[0].astype(jnp.int32)
    src = edge_index[1].astype(jnp.int32)
    w = edge_weight.astype(jnp.float32)

    eps = e // NS
    assert eps * NS == e
    chunk = next(c for c in range(128, 7, -8) if eps % c == 0)
    nch = eps // chunk

    feat2 = features.reshape(2 * n, DH)
    src3d = src.reshape(NS, nch, chunk)
    dst3d = dst.reshape(NS, nch, chunk)
    w3d = w.reshape(NS, nch, chunk)

    partials = _sc_aggregate(feat2, src3d, dst3d, w3d, n)
    return _tc_linear(partials, W, b, n)


# speedup vs baseline: 9.7858x; 1.0391x over previous
import dataclasses
import functools

import jax
import jax.numpy as jnp
from jax import lax
from jax.experimental import pallas as pl
from jax.experimental.pallas import tpu as pltpu
from jax.experimental.pallas import tpu_sc as plsc

NC = 2
NS = 16
DH = 64
VPR = DH // 16


def _sc_aggregate(feat2, src3d, dst3d, w3d, n):
    nch, chunk = src3d.shape[1:]
    zrows = chunk * 8
    npad = NS * zrows
    assert npad >= n

    mesh = plsc.VectorSubcoreMesh(
        core_axis_name="c", subcore_axis_name="s", num_cores=NC, num_subcores=NS
    )
    cp = pltpu.CompilerParams()
    if "needs_layout_passes" in pltpu.CompilerParams.__dataclass_fields__:
        cp = dataclasses.replace(cp, needs_layout_passes=False)
    if "use_tc_tiling_on_sc" in pltpu.CompilerParams.__dataclass_fields__:
        cp = dataclasses.replace(cp, use_tc_tiling_on_sc=False)

    nbuf = 4
    main_end = max(0, ((nch - 2) // nbuf) * nbuf)

    @functools.partial(
        pl.kernel,
        out_type=jax.ShapeDtypeStruct((NC, npad, DH), jnp.float32),
        mesh=mesh,
        compiler_params=cp,
        scratch_types=[
            pltpu.VMEM((nch, chunk), jnp.int32),
            pltpu.VMEM((nch, chunk), jnp.int32),
            pltpu.VMEM((nch * chunk,), jnp.float32),
            [pltpu.VMEM((chunk, DH), jnp.float32)] * nbuf,
            [pltpu.SemaphoreType.DMA] * nbuf,
            [pltpu.SemaphoreType.DMA] * nbuf,
            pltpu.VMEM_SHARED((npad, DH), jnp.float32),
        ],
    )
    def agg_kernel(feat_hbm, src_hbm, dst_hbm, w_hbm, out_hbm,
                   src_v, dst_v, w_v, bufs, gsems, ssems, acc):
        cid = lax.axis_index("c")
        sid = lax.axis_index("s")

        pltpu.sync_copy(src_hbm.at[sid], src_v)
        pltpu.sync_copy(dst_hbm.at[sid], dst_v)
        pltpu.sync_copy(w_hbm.at[sid], w_v)

        @pl.loop(0, nch)
        def _(r):
            for k in range(chunk // 16):
                sl = pl.ds(k * 16, 16)
                src_v[r, sl] = src_v[r, sl] * 2 + cid

        zero = jnp.zeros((16,), jnp.float32)

        @pl.loop(0, chunk)
        def _(r):
            for k in range(VPR):
                bufs[0][r, pl.ds(k * 16, 16)] = zero

        for i in range(8):
            pltpu.sync_copy(bufs[0], acc.at[pl.ds(sid * zrows + i * chunk, chunk)])
        plsc.subcore_barrier()

        def gather_start(t, j):
            pltpu.async_copy(feat_hbm.at[src_v.at[t]], bufs[j], gsems[j])

        def gather_wait(t, j):
            pltpu.make_async_copy(feat_hbm.at[src_v.at[t]], bufs[j], gsems[j]).wait()

        def scatter_start(t, j):
            pltpu.async_copy(bufs[j], acc.at[dst_v.at[t]], ssems[j], add=True)

        def scatter_wait(t, j):
            pltpu.make_async_copy(bufs[j], acc.at[dst_v.at[t]], ssems[j]).wait()

        def scale(t, j):
            buf = bufs[j]
            base = jnp.full((16,), t * chunk, jnp.int32)

            @pl.loop(0, chunk, unroll=4)
            def _(r):
                wj = plsc.load_gather(w_v, [base + r])
                for k in range(VPR):
                    sl = pl.ds(k * 16, 16)
                    buf[r, sl] = buf[r, sl] * wj

        gather_start(0, 0)
        gather_start(1, 1)

        @pl.loop(0, main_end, step=nbuf)
        def _(t0):
            for j in range(nbuf):
                t = t0 + j
                j2 = (j + 2) % nbuf

                @pl.when(t >= 2)
                def _():
                    scatter_wait(t - 2, j2)

                @pl.when(t + 2 < nch)
                def _():
                    gather_start(t + 2, j2)

                gather_wait(t, j)
                scale(t, j)
                scatter_start(t, j)

        for t in range(main_end, nch):
            j = t % nbuf
            if t >= 2:
                scatter_wait(t - 2, (t + 2) % nbuf)
            if t + 2 < nch:
                gather_start(t + 2, (t + 2) % nbuf)
            gather_wait(t, j)
            scale(t, j)
            scatter_start(t, j)

        scatter_wait(nch - 2, (nch - 2) % nbuf)
        scatter_wait(nch - 1, (nch - 1) % nbuf)
        plsc.subcore_barrier()

        pltpu.sync_copy(
            acc.at[pl.ds(sid * zrows, zrows)],
            out_hbm.at[cid, pl.ds(sid * zrows, zrows)],
        )

    return agg_kernel(feat2, src3d, dst3d, w3d)


def _linear_body(p_ref, w_ref, b_ref, o_ref):
    y0 = lax.dot_general(
        p_ref[0], w_ref[:, 0:DH], (((1,), (1,)), ((), ())),
        preferred_element_type=jnp.float32,
    )
    y1 = lax.dot_general(
        p_ref[1], w_ref[:, DH:2 * DH], (((1,), (1,)), ((), ())),
        preferred_element_type=jnp.float32,
    )
    o_ref[...] = y0 + y1 + b_ref[...]


def _tc_linear(partials, W, b, n):
    nc, npad, dh = partials.shape
    d_out = W.shape[0]
    blk = 1000
    assert n % blk == 0
    return pl.pallas_call(
        _linear_body,
        grid=(n // blk,),
        in_specs=[
            pl.BlockSpec((nc, blk, dh), lambda i: (0, i, 0)),
            pl.BlockSpec((d_out, 2 * dh), lambda i: (0, 0)),
            pl.BlockSpec((1, d_out), lambda i: (0, 0)),
        ],
        out_specs=pl.BlockSpec((blk, d_out), lambda i: (i, 0)),
        out_shape=jax.ShapeDtypeStruct((n, d_out), jnp.float32),
    )(partials, W, b.reshape(1, d_out))


def kernel(features, edge_index, edge_weight, W, b):
    n, d = features.shape
    e = edge_weight.shape[0]
    assert d == 2 * DH
    dst = edge_index[0].astype(jnp.int32)
    src = edge_index[1].astype(jnp.int32)
    w = edge_weight.astype(jnp.float32)

    eps = e // NS
    assert eps * NS == e
    chunk = next(c for c in range(128, 7, -8) if eps % c == 0)
    nch = eps // chunk

    feat2 = features.reshape(2 * n, DH)
    src3d = src.reshape(NS, nch, chunk)
    dst3d = dst.reshape(NS, nch, chunk)
    w3d = w.reshape(NS, eps)

    partials = _sc_aggregate(feat2, src3d, dst3d, w3d, n)
    return _tc_linear(partials, W, b, n)
